# baseline (speedup 1.0000x reference)
import jax
import jax.numpy as jnp
from jax.experimental import pallas as pl
from jax.experimental.pallas import tpu as pltpu

E = 64
H = 768
I = 3072


NSPLIT = 8
CHUNK = I // NSPLIT


def _ffn_block_kernel(x_ref, b1_ref, b2_ref, *w_and_o):
    w_refs = w_and_o[:-1]
    o_ref = w_and_o[-1]
    e = pl.program_id(0)
    per = x_ref.shape[0] // pl.num_programs(0)
    x = x_ref[pl.ds(e * per, per), :]
    o = b2_ref[pl.ds(e, 1), :]
    for q in range(NSPLIT):
        w1q = w_refs[2 * q]
        w2q = w_refs[2 * q + 1]
        h = jnp.dot(x, w1q[0], preferred_element_type=jnp.float32)
        h = h + b1_ref[pl.ds(e, 1), q * CHUNK:(q + 1) * CHUNK]
        o = o + jnp.dot(h, w2q[0], preferred_element_type=jnp.float32)
    o_ref[pl.ds(e * per, per), :] = o


def kernel(hidden_states, W1, b1, W2, b2):
    Bb, Ss, Hh = hidden_states.shape
    Ee = W1.shape[0]
    T = Bb * Ss
    x = hidden_states.reshape(T, Hh)

    w_specs = []
    w_args = []
    for q in range(NSPLIT):
        w_specs.append(
            pl.BlockSpec((1, Hh, CHUNK), lambda e, q=q: (e, 0, q)))
        w_args.append(W1)
        w_specs.append(
            pl.BlockSpec((1, CHUNK, Hh), lambda e, q=q: (e, q, 0)))
        w_args.append(W2)

    out = pl.pallas_call(
        _ffn_block_kernel,
        grid=(Ee,),
        in_specs=[
            pl.BlockSpec((T, Hh), lambda e: (0, 0)),
            pl.BlockSpec((Ee, I), lambda e: (0, 0)),
            pl.BlockSpec((Ee, Hh), lambda e: (0, 0)),
        ] + w_specs,
        out_specs=pl.BlockSpec((T, Hh), lambda e: (0, 0)),
        out_shape=jax.ShapeDtypeStruct((T, Hh), jnp.float32),
        compiler_params=pltpu.CompilerParams(
            dimension_semantics=("arbitrary",),
        ),
    )(x, b1, b2, *w_args)
    return out.reshape(Bb, Ss, Hh)


# speedup vs baseline: 1.0038x; 1.0038x over previous
import jax
import jax.numpy as jnp
from jax.experimental import pallas as pl
from jax.experimental.pallas import tpu as pltpu

E = 64
H = 768
I = 3072
PER = 32
NSPLIT = 4
CHUNK = I // NSPLIT
NBUF = 8
S = E * NSPLIT


def _ffn_manual_kernel(x_ref, b1_ref, b2_ref, w1_hbm, w2_hbm, o_ref,
                       w1_buf, w2_buf, sem1, sem2):
    def w1_copy(s, slot):
        e = s // NSPLIT
        q = s % NSPLIT
        return pltpu.make_async_copy(
            w1_hbm.at[e, :, pl.ds(q * CHUNK, CHUNK)],
            w1_buf.at[slot], sem1.at[slot])

    def w2_copy(s, slot):
        e = s // NSPLIT
        q = s % NSPLIT
        return pltpu.make_async_copy(
            w2_hbm.at[e, pl.ds(q * CHUNK, CHUNK), :],
            w2_buf.at[slot], sem2.at[slot])

    for s0 in range(NBUF):
        w1_copy(s0, s0).start()
        w2_copy(s0, s0).start()

    def step(s, carry):
        slot = jax.lax.rem(s, NBUF)
        e = s // NSPLIT
        q = s % NSPLIT
        w1_copy(s, slot).wait()
        w2_copy(s, slot).wait()
        x = x_ref[pl.ds(e * PER, PER), :]
        h = jnp.dot(x, w1_buf[slot], preferred_element_type=jnp.float32)
        h = h + b1_ref[pl.ds(e, 1), pl.ds(q * CHUNK, CHUNK)]
        part = jnp.dot(h, w2_buf[slot], preferred_element_type=jnp.float32)

        @pl.when(q == 0)
        def _init():
            o_ref[pl.ds(e * PER, PER), :] = part + b2_ref[pl.ds(e, 1), :]

        @pl.when(q != 0)
        def _acc():
            o_ref[pl.ds(e * PER, PER), :] += part

        @pl.when(s + NBUF < S)
        def _prefetch():
            w1_copy(s + NBUF, slot).start()
            w2_copy(s + NBUF, slot).start()

        return carry

    jax.lax.fori_loop(0, S, step, 0)


def kernel(hidden_states, W1, b1, W2, b2):
    Bb, Ss, Hh = hidden_states.shape
    T = Bb * Ss
    x = hidden_states.reshape(T, Hh)

    out = pl.pallas_call(
        _ffn_manual_kernel,
        in_specs=[
            pl.BlockSpec(memory_space=pltpu.MemorySpace.VMEM),
            pl.BlockSpec(memory_space=pltpu.MemorySpace.VMEM),
            pl.BlockSpec(memory_space=pltpu.MemorySpace.VMEM),
            pl.BlockSpec(memory_space=pltpu.MemorySpace.HBM),
            pl.BlockSpec(memory_space=pltpu.MemorySpace.HBM),
        ],
        out_specs=pl.BlockSpec(memory_space=pltpu.MemorySpace.VMEM),
        out_shape=jax.ShapeDtypeStruct((T, Hh), jnp.float32),
        scratch_shapes=[
            pltpu.VMEM((NBUF, H, CHUNK), jnp.float32),
            pltpu.VMEM((NBUF, CHUNK, H), jnp.float32),
            pltpu.SemaphoreType.DMA((NBUF,)),
            pltpu.SemaphoreType.DMA((NBUF,)),
        ],
        compiler_params=pltpu.CompilerParams(
            vmem_limit_bytes=112 * 1024 * 1024,
        ),
    )(x, b1, b2, W1, W2)
    return out.reshape(Bb, Ss, Hh)


# speedup vs baseline: 1.0054x; 1.0016x over previous
import jax
import jax.numpy as jnp
from jax.experimental import pallas as pl
from jax.experimental.pallas import tpu as pltpu

E = 64
H = 768
I = 3072
NS1 = 3
NS2 = 3
C1 = H // NS1
C2 = I // NS2


def _ffn_block_kernel(x_ref, b1_ref, b2_ref, *w_and_o):
    w1_refs = w_and_o[:NS1]
    w2_refs = w_and_o[NS1:NS1 + NS2]
    o_ref = w_and_o[-1]
    e = pl.program_id(0)
    per = x_ref.shape[0] // pl.num_programs(0)
    x = x_ref[pl.ds(e * per, per), :]
    h = b1_ref[pl.ds(e, 1), :]
    for r in range(NS1):
        h = h + jnp.dot(x[:, r * C1:(r + 1) * C1], w1_refs[r][0],
                        preferred_element_type=jnp.float32)
    o = b2_ref[pl.ds(e, 1), :]
    for q in range(NS2):
        o = o + jnp.dot(h[:, q * C2:(q + 1) * C2], w2_refs[q][0],
                        preferred_element_type=jnp.float32)
    o_ref[pl.ds(e * per, per), :] = o


def kernel(hidden_states, W1, b1, W2, b2):
    Bb, Ss, Hh = hidden_states.shape
    Ee = W1.shape[0]
    T = Bb * Ss
    x = hidden_states.reshape(T, Hh)

    w_specs = []
    w_args = []
    for r in range(NS1):
        w_specs.append(pl.BlockSpec((1, C1, I), lambda e, r=r: (e, r, 0)))
        w_args.append(W1)
    for q in range(NS2):
        w_specs.append(pl.BlockSpec((1, C2, Hh), lambda e, q=q: (e, q, 0)))
        w_args.append(W2)

    out = pl.pallas_call(
        _ffn_block_kernel,
        grid=(Ee,),
        in_specs=[
            pl.BlockSpec((T, Hh), lambda e: (0, 0)),
            pl.BlockSpec((Ee, I), lambda e: (0, 0)),
            pl.BlockSpec((Ee, Hh), lambda e: (0, 0)),
        ] + w_specs,
        out_specs=pl.BlockSpec((T, Hh), lambda e: (0, 0)),
        out_shape=jax.ShapeDtypeStruct((T, Hh), jnp.float32),
        compiler_params=pltpu.CompilerParams(
            dimension_semantics=("arbitrary",),
        ),
    )(x, b1, b2, *w_args)
    return out.reshape(Bb, Ss, Hh)


# speedup vs baseline: 1.0057x; 1.0003x over previous
import jax
import jax.numpy as jnp
from jax.experimental import pallas as pl
from jax.experimental.pallas import tpu as pltpu

E = 64
H = 768
I = 3072


NSPLIT = 4
CHUNK = I // NSPLIT


def _ffn_block_kernel(x_ref, b1_ref, b2_ref, *w_and_o):
    w_refs = w_and_o[:-1]
    o_ref = w_and_o[-1]
    e = pl.program_id(0)
    per = x_ref.shape[0] // pl.num_programs(0)
    x = x_ref[pl.ds(e * per, per), :]
    o = b2_ref[pl.ds(e, 1), :]
    for q in range(NSPLIT):
        w1q = w_refs[2 * q]
        w2q = w_refs[2 * q + 1]
        h = jnp.dot(x, w1q[0], preferred_element_type=jnp.float32)
        h = h + b1_ref[pl.ds(e, 1), q * CHUNK:(q + 1) * CHUNK]
        o = o + jnp.dot(h, w2q[0], preferred_element_type=jnp.float32)
    o_ref[0] = o


def kernel(hidden_states, W1, b1, W2, b2):
    Bb, Ss, Hh = hidden_states.shape
    Ee = W1.shape[0]
    T = Bb * Ss
    x = hidden_states.reshape(T, Hh)

    w_specs = []
    w_args = []
    for q in range(NSPLIT):
        w_specs.append(
            pl.BlockSpec((1, Hh, CHUNK), lambda e, q=q: (e, 0, q)))
        w_args.append(W1)
        w_specs.append(
            pl.BlockSpec((1, CHUNK, Hh), lambda e, q=q: (e, q, 0)))
        w_args.append(W2)

    out = pl.pallas_call(
        _ffn_block_kernel,
        grid=(Ee,),
        in_specs=[
            pl.BlockSpec((T, Hh), lambda e: (0, 0)),
            pl.BlockSpec((Ee, I), lambda e: (0, 0)),
            pl.BlockSpec((Ee, Hh), lambda e: (0, 0)),
        ] + w_specs,
        out_specs=pl.BlockSpec((1, T // Ee, Hh), lambda e: (e, 0, 0)),
        out_shape=jax.ShapeDtypeStruct((Ee, T // Ee, Hh), jnp.float32),
        compiler_params=pltpu.CompilerParams(
            dimension_semantics=("arbitrary",),
        ),
    )(x, b1, b2, *w_args)
    return out.reshape(Bb, Ss, Hh)


# speedup vs baseline: 1.0068x; 1.0011x over previous
import jax
import jax.numpy as jnp
from jax.experimental import pallas as pl
from jax.experimental.pallas import tpu as pltpu

E = 64
H = 768
I = 3072


NSPLIT = 6
CHUNK = I // NSPLIT


def _ffn_block_kernel(x_ref, b1_ref, b2_ref, *w_and_o):
    w_refs = w_and_o[:-1]
    o_ref = w_and_o[-1]
    e = pl.program_id(0)
    per = x_ref.shape[0] // pl.num_programs(0)
    x = x_ref[pl.ds(e * per, per), :]
    o = b2_ref[pl.ds(e, 1), :]
    for q in range(NSPLIT):
        w1q = w_refs[2 * q]
        w2q = w_refs[2 * q + 1]
        h = jnp.dot(x, w1q[0], preferred_element_type=jnp.float32)
        h = h + b1_ref[pl.ds(e, 1), q * CHUNK:(q + 1) * CHUNK]
        o = o + jnp.dot(h, w2q[0], preferred_element_type=jnp.float32)
    o_ref[0] = o


def kernel(hidden_states, W1, b1, W2, b2):
    Bb, Ss, Hh = hidden_states.shape
    Ee = W1.shape[0]
    T = Bb * Ss
    x = hidden_states.reshape(T, Hh)

    w_specs = []
    w_args = []
    for q in range(NSPLIT):
        w_specs.append(
            pl.BlockSpec((1, Hh, CHUNK), lambda e, q=q: (e, 0, q)))
        w_args.append(W1)
        w_specs.append(
            pl.BlockSpec((1, CHUNK, Hh), lambda e, q=q: (e, q, 0)))
        w_args.append(W2)

    out = pl.pallas_call(
        _ffn_block_kernel,
        grid=(Ee,),
        in_specs=[
            pl.BlockSpec((T, Hh), lambda e: (0, 0)),
            pl.BlockSpec((Ee, I), lambda e: (0, 0)),
            pl.BlockSpec((Ee, Hh), lambda e: (0, 0)),
        ] + w_specs,
        out_specs=pl.BlockSpec((1, T // Ee, Hh), lambda e: (e, 0, 0)),
        out_shape=jax.ShapeDtypeStruct((Ee, T // Ee, Hh), jnp.float32),
        compiler_params=pltpu.CompilerParams(
            dimension_semantics=("arbitrary",),
        ),
    )(x, b1, b2, *w_args)
    return out.reshape(Bb, Ss, Hh)
